# baseline (speedup 1.0000x reference)
import functools

import jax
import jax.numpy as jnp
from jax import lax
from jax.experimental import pallas as pl
from jax.experimental.pallas import tpu as pltpu
from jax.experimental.pallas import tpu_sc as plsc

NUM_CORES = 2
NUM_SUBCORES = 16
LANES = 16
NUM_WORKERS = NUM_CORES * NUM_SUBCORES

BATCH = 16384
FACTORS = 64
PADDED = 2 * FACTORS
B_PER_W = BATCH // NUM_WORKERS
CHUNK = 256
N_CHUNKS = B_PER_W // CHUNK
GLOBAL_MEAN = 3.5


@functools.partial(
    pl.kernel,
    out_type=jax.ShapeDtypeStruct((BATCH,), jnp.float32),
    mesh=plsc.VectorSubcoreMesh(core_axis_name="c", subcore_axis_name="s"),
    compiler_params=pltpu.CompilerParams(needs_layout_passes=False,
                                         use_tc_tiling_on_sc=True),
    scratch_types=[
        pltpu.VMEM((B_PER_W,), jnp.int32),
        pltpu.VMEM((B_PER_W,), jnp.int32),
        pltpu.VMEM((B_PER_W,), jnp.int32),
        pltpu.VMEM((B_PER_W,), jnp.int32),
        pltpu.VMEM((B_PER_W,), jnp.int32),
        pltpu.VMEM((B_PER_W,), jnp.int32),
        pltpu.VMEM((CHUNK, PADDED), jnp.float32),
        pltpu.VMEM((CHUNK, PADDED), jnp.float32),
        pltpu.VMEM((B_PER_W,), jnp.float32),
        pltpu.VMEM((B_PER_W,), jnp.float32),
        pltpu.VMEM((B_PER_W,), jnp.float32),
        pltpu.SemaphoreType.DMA,
        pltpu.SemaphoreType.DMA,
        pltpu.SemaphoreType.DMA,
        pltpu.SemaphoreType.DMA,
    ],
)
def _mf_sc_kernel(users_hbm, items_hbm, uemb_hbm, iemb_hbm, ubias_hbm,
                  ibias_hbm, out_hbm, uidx_v, iidx_v, urow_v, irow_v,
                  uoff_v, ioff_v, urows_v, irows_v, ub_v, ib_v, out_v,
                  sem_u, sem_i, sem_ub, sem_ib):
    wid = lax.axis_index("s") * NUM_CORES + lax.axis_index("c")
    base = wid * B_PER_W

    pltpu.sync_copy(users_hbm.at[pl.ds(base, B_PER_W)], uidx_v)
    pltpu.sync_copy(items_hbm.at[pl.ds(base, B_PER_W)], iidx_v)

    cub = pltpu.async_copy(ubias_hbm.at[uidx_v], ub_v, sem_ub)
    cib = pltpu.async_copy(ibias_hbm.at[iidx_v], ib_v, sem_ib)

    def prep_body(j, carry):
        sl = pl.ds(j * LANES, LANES)
        u = uidx_v[sl]
        urow_v[sl] = lax.shift_right_logical(u, 1)
        uoff_v[sl] = lax.shift_left(jnp.bitwise_and(u, 1), 6)
        i = iidx_v[sl]
        irow_v[sl] = lax.shift_right_logical(i, 1)
        ioff_v[sl] = lax.shift_left(jnp.bitwise_and(i, 1), 6)
        return carry

    lax.fori_loop(0, B_PER_W // LANES, prep_body, 0)

    cub.wait()
    cib.wait()

    lane = lax.iota(jnp.int32, LANES)

    for c in range(N_CHUNKS):
        cbase = c * CHUNK
        cu = pltpu.async_copy(
            uemb_hbm.at[urow_v.at[pl.ds(cbase, CHUNK)]], urows_v, sem_u)
        ci = pltpu.async_copy(
            iemb_hbm.at[irow_v.at[pl.ds(cbase, CHUNK)]], irows_v, sem_i)
        cu.wait()
        ci.wait()

        def group_body(g, carry):
            gbase = cbase + g * LANES
            rows = g * LANES + lane
            uoff = uoff_v[pl.ds(gbase, LANES)]
            ioff = ioff_v[pl.ds(gbase, LANES)]
            acc = (ub_v[pl.ds(gbase, LANES)] + ib_v[pl.ds(gbase, LANES)]
                   + GLOBAL_MEAN)
            for k in range(FACTORS):
                u = plsc.load_gather(urows_v, [rows, uoff + k])
                v = plsc.load_gather(irows_v, [rows, ioff + k])
                acc = acc + u * v
            out_v[pl.ds(gbase, LANES)] = acc
            return carry

        lax.fori_loop(0, CHUNK // LANES, group_body, 0)

    pltpu.sync_copy(out_v, out_hbm.at[pl.ds(base, B_PER_W)])


def kernel(users, items, user_emb, item_emb, user_bias, item_bias):
    uemb2 = user_emb.reshape(-1, PADDED)
    iemb2 = item_emb.reshape(-1, PADDED)
    return _mf_sc_kernel(users.astype(jnp.int32), items.astype(jnp.int32),
                         uemb2, iemb2,
                         user_bias.reshape(-1), item_bias.reshape(-1))
